# baseline (speedup 1.0000x reference)
import jax
import jax.numpy as jnp
from jax.experimental import pallas as pl
from jax.experimental.pallas import tpu as pltpu


def _body(t_hbm, o_hbm, buf, sem_in, sem_out):
    load = pltpu.make_async_copy(t_hbm, buf, sem_in)
    load.start()
    load.wait()
    copies = [
        pltpu.make_async_copy(buf, o_hbm.at[b], sem_out.at[b])
        for b in range(o_hbm.shape[0])
    ]
    for c in copies:
        c.start()
    for c in copies:
        c.wait()


def kernel(x, pos_table):
    B = x.shape[0]
    P, D = pos_table.shape
    return pl.pallas_call(
        _body,
        in_specs=[pl.BlockSpec(memory_space=pl.ANY)],
        out_specs=pl.BlockSpec(memory_space=pl.ANY),
        out_shape=jax.ShapeDtypeStruct((B, P, D), jnp.float32),
        scratch_shapes=[
            pltpu.VMEM((P, D), jnp.float32),
            pltpu.SemaphoreType.DMA,
            pltpu.SemaphoreType.DMA((B,)),
        ],
    )(pos_table)


# speedup vs baseline: 4.6605x; 4.6605x over previous
import jax
import jax.numpy as jnp
from jax.experimental import pallas as pl
from jax.experimental.pallas import tpu as pltpu


def _body(t_hbm, o_hbm, buf, sem_in, sem_out):
    load = pltpu.make_async_copy(t_hbm, buf, sem_in)
    load.start()
    load.wait()
    copies = [
        pltpu.make_async_copy(buf, o_hbm.at[b], sem_out.at[b])
        for b in range(o_hbm.shape[0])
    ]
    for c in copies:
        c.start()
    for c in copies:
        c.wait()


def kernel(x, pos_table):
    B = x.shape[0]
    P, D = pos_table.shape
    table_t = pos_table.T
    out_t = pl.pallas_call(
        _body,
        in_specs=[pl.BlockSpec(memory_space=pl.ANY)],
        out_specs=pl.BlockSpec(memory_space=pl.ANY),
        out_shape=jax.ShapeDtypeStruct((B, D, P), jnp.float32),
        scratch_shapes=[
            pltpu.VMEM((D, P), jnp.float32),
            pltpu.SemaphoreType.DMA,
            pltpu.SemaphoreType.DMA((B,)),
        ],
    )(table_t)
    return jnp.transpose(out_t, (0, 2, 1))
